# baseline (speedup 1.0000x reference)
import functools

import jax
import jax.numpy as jnp
from jax import lax
from jax.experimental import pallas as pl
from jax.experimental.pallas import tpu as pltpu
from jax.experimental.pallas import tpu_sc as plsc

N = 10000
HID = 128
H = 8
C = 16
NP = 10240
NBLK = NP // 256
NW = 32
K1 = 1296
K2 = 2592

@functools.lru_cache(maxsize=1)
def _mesh():
    return plsc.VectorSubcoreMesh(
        core_axis_name="c", subcore_axis_name="s",
        num_cores=2, num_subcores=16)


def _f32(shape):
    return jax.ShapeDtypeStruct(shape, jnp.float32)


def _proj_body(xt, wt, pb, out):
    out[...] = jnp.dot(wt[...], xt[...], preferred_element_type=jnp.float32) \
        + pb[...].T


def _proj(xT, Wt, pb2d):
    return pl.pallas_call(
        _proj_body,
        grid=(NBLK,),
        in_specs=[
            pl.BlockSpec((HID, 256), lambda i: (0, i)),
            pl.BlockSpec((HID, HID), lambda i: (0, 0)),
            pl.BlockSpec((1, HID), lambda i: (0, 0)),
        ],
        out_specs=pl.BlockSpec((HID, 256), lambda i: (0, i)),
        out_shape=_f32((HID, NP)),
    )(xT, Wt, pb2d)


def _tc1_body(act, wt, asb, adb, xwt, asrc, adst, ms, md):
    i = pl.program_id(0)
    a = act[...]
    xw = jnp.dot(wt[...], a, preferred_element_type=jnp.float32)
    xwt[...] = xw
    s = jnp.dot(asb[...], xw, preferred_element_type=jnp.float32)
    d = jnp.dot(adb[...], xw, preferred_element_type=jnp.float32)
    asrc[...] = s
    adst[...] = d
    neg = jnp.full((H, 128), -jnp.inf, jnp.float32)
    pms = jnp.where(i == 0, neg, ms[...])
    pmd = jnp.where(i == 0, neg, md[...])
    ms[...] = jnp.maximum(pms, jnp.max(s, axis=1)[:, None])
    md[...] = jnp.maximum(pmd, jnp.max(d, axis=1)[:, None])


def _tc1(actT, Wt, As, Ad):
    return pl.pallas_call(
        _tc1_body,
        grid=(NBLK,),
        in_specs=[
            pl.BlockSpec((HID, 256), lambda i: (0, i)),
            pl.BlockSpec((HID, HID), lambda i: (0, 0)),
            pl.BlockSpec((H, HID), lambda i: (0, 0)),
            pl.BlockSpec((H, HID), lambda i: (0, 0)),
        ],
        out_specs=[
            pl.BlockSpec((HID, 256), lambda i: (0, i)),
            pl.BlockSpec((H, 256), lambda i: (0, i)),
            pl.BlockSpec((H, 256), lambda i: (0, i)),
            pl.BlockSpec((H, 128), lambda i: (0, 0)),
            pl.BlockSpec((H, 128), lambda i: (0, 0)),
        ],
        out_shape=[_f32((HID, NP)), _f32((H, NP)), _f32((H, NP)),
                   _f32((H, 128)), _f32((H, 128))],
    )(actT, Wt, As, Ad)


def _cls_body(tut, tdt, semp, w1t, b1p, w2t, b2p, out):
    s0 = semp[0, 0]
    s1 = semp[0, 1]
    m = jnp.maximum(s0, s1)
    e0 = jnp.exp(s0 - m)
    e1 = jnp.exp(s1 - m)
    w0 = e0 / (e0 + e1)
    w1 = e1 / (e0 + e1)
    z = w0 * tut[...] + w1 * tdt[...]
    hh = jnp.dot(w1t[...], z, preferred_element_type=jnp.float32) \
        + b1p[...][:, :64].T
    hh = jnp.maximum(hh, 0.0)
    lg = jnp.dot(w2t[...][:, :64], hh, preferred_element_type=jnp.float32) \
        + b2p[...][:, :8].T
    out[...] = lg


def _cls(tutT, tdtT, semp, W1t, b1p, W2t8, b2p):
    return pl.pallas_call(
        _cls_body,
        grid=(NBLK,),
        in_specs=[
            pl.BlockSpec((HID, 256), lambda i: (0, i)),
            pl.BlockSpec((HID, 256), lambda i: (0, i)),
            pl.BlockSpec((1, 128), lambda i: (0, 0)),
            pl.BlockSpec((64, 128), lambda i: (0, 0)),
            pl.BlockSpec((1, 128), lambda i: (0, 0)),
            pl.BlockSpec((8, 128), lambda i: (0, 0)),
            pl.BlockSpec((1, 128), lambda i: (0, 0)),
        ],
        out_specs=pl.BlockSpec((8, 256), lambda i: (0, i)),
        out_shape=_f32((8, NP)),
    )(tutT, tdtT, semp, W1t, b1p, W2t8, b2p)


def _splat_i32(v):
    return jnp.zeros((16,), jnp.int32) + v


def _p2_builder(Ep):
    nch = Ep // K2

    def body(pkH, asrcF, adstF, g16H, xwFH, biasH, outFH,
             xw4, out4, asv, adv, denv, bv, pkb, g16v, sem0, sem1):
        cid = lax.axis_index("c")
        sid = lax.axis_index("s")
        wid = sid * 2 + cid
        hd = wid // 4
        pltpu.sync_copy(xwFH.at[pl.ds(4 * wid * NP, 4 * NP)], xw4)
        pltpu.sync_copy(asrcF.at[pl.ds(hd * NP, NP)], asv)
        pltpu.sync_copy(adstF.at[pl.ds(hd * NP, NP)], adv)
        pltpu.sync_copy(biasH, bv)
        pltpu.sync_copy(g16H, g16v)
        iota = lax.iota(jnp.int32, 16)
        gh = plsc.load_gather(g16v, [_splat_i32(0) + hd])

        @plsc.parallel_loop(0, 4 * NP // 16, 1, unroll=8)
        def _(t):
            idx = _splat_i32(t * 16) + iota
            plsc.store_scatter(out4, [idx], jnp.zeros((16,), jnp.float32))

        @plsc.parallel_loop(0, NP // 16, 1, unroll=8)
        def _(t):
            idx = _splat_i32(t * 16) + iota
            plsc.store_scatter(denv, [idx], jnp.zeros((16,), jnp.float32))

        sems = (sem0, sem1)

        def issue(ci, b):
            pltpu.async_copy(pkH.at[pl.ds(ci * K2, K2)], pkb.at[b], sems[b])

        issue(0, 0)
        issue(1, 1)

        def obody(i2, _):
            for b in (0, 1):
                ci = i2 * 2 + b
                pltpu.make_async_copy(
                    pkH.at[pl.ds(ci * K2, K2)], pkb.at[b], sems[b]).wait()
                bvec = _splat_i32(b)

                @plsc.parallel_loop(0, K2 // 16, 1, unroll=1)
                def _(g):
                    lidx = _splat_i32(g * 16) + iota
                    pk = plsc.load_gather(pkb, [bvec, lidx])
                    sv = pk & 16383
                    dv = pk >> 14
                    al = plsc.load_gather(asv, [sv]) \
                        + plsc.load_gather(adv, [dv])
                    al = jnp.maximum(al, 0.2 * al) - gh
                    ex = jnp.exp(al)
                    plsc.addupdate_scatter(denv, [dv], ex)
                    for c in range(4):
                        cN = _splat_i32(c * NP)
                        xg = plsc.load_gather(xw4, [cN + sv])
                        plsc.addupdate_scatter(out4, [cN + dv], xg * ex)

                @pl.when(ci + 2 < nch)
                def _():
                    issue(ci + 2, b)
            return 0
        lax.fori_loop(0, nch // 2, obody, 0)

        for c in range(4):
            bcv = plsc.load_gather(bv, [_splat_i32(0) + 4 * wid + c])

            @plsc.parallel_loop(0, NP // 16, 1, unroll=8)
            def _(t):
                tidx = _splat_i32(t * 16) + iota
                idx = _splat_i32(c * NP) + tidx
                den = plsc.load_gather(denv, [tidx]) + 1e-16
                v = plsc.load_gather(out4, [idx]) / den + bcv
                act = jnp.where(v > 0, v, jnp.exp(v) - 1.0)
                plsc.store_scatter(out4, [idx], act)
        pltpu.sync_copy(out4, outFH.at[pl.ds(4 * wid * NP, 4 * NP)])

    return pl.kernel(
        body,
        out_type=_f32((HID * NP,)),
        mesh=_mesh(),
        compiler_params=pltpu.CompilerParams(
            use_tc_tiling_on_sc=False, needs_layout_passes=False),
        scratch_types=[
            pltpu.VMEM((4 * NP,), jnp.float32),
            pltpu.VMEM((4 * NP,), jnp.float32),
            pltpu.VMEM((NP,), jnp.float32),
            pltpu.VMEM((NP,), jnp.float32),
            pltpu.VMEM((NP,), jnp.float32),
            pltpu.VMEM((HID,), jnp.float32),
            pltpu.VMEM((2, K2), jnp.int32),
            pltpu.VMEM((16,), jnp.float32),
            pltpu.SemaphoreType.DMA,
            pltpu.SemaphoreType.DMA,
        ],
    )


def _blockdiag(att):
    return (att[:, None, :] * jnp.eye(H, dtype=att.dtype)[:, :, None]
            ).reshape(H, H * C)


def _gat_layer(actT, W, att_s, att_d, bias, pkA, p2):
    xwT, asrcT, adstT, mS, mD = _tc1(
        actT, W.T, _blockdiag(att_s), _blockdiag(att_d))
    sm = mS[:, 0] + mD[:, 0]
    g16 = jnp.concatenate([jnp.maximum(sm, 0.2 * sm)] * 2)
    outF = p2(pkA, asrcT.reshape(-1), adstT.reshape(-1), g16,
              xwT.reshape(-1), bias)
    return outF.reshape(HID, NP)


def kernel(x, proj_W, proj_b,
           tut_W1, tut_as1, tut_ad1, tut_b1, tut_W2, tut_as2, tut_ad2, tut_b2,
           tdt_W1, tdt_as1, tdt_ad1, tdt_b1, tdt_W2, tdt_as2, tdt_ad2, tdt_b2,
           sem, cls_W1, cls_b1, cls_W2, cls_b2,
           tut_edge_index, tdt_edge_index):
    n = x.shape[0]
    e = tut_edge_index.shape[1]
    etot = e + n
    npt = -(-etot // (NW * K1)) * K1
    Ep = NW * npt

    xT = jnp.pad(x.T, ((0, 0), (0, NP - n)))
    hT = _proj(xT, proj_W.T, proj_b[None, :])

    p2 = _p2_builder(Ep)

    loop = jnp.arange(n, dtype=jnp.int32)
    pad = jnp.full((Ep - etot,), n, jnp.int32)

    acts = {}
    for ei, params in (
            (tut_edge_index,
             ((tut_W1, tut_as1, tut_ad1, tut_b1),
              (tut_W2, tut_as2, tut_ad2, tut_b2))),
            (tdt_edge_index,
             ((tdt_W1, tdt_as1, tdt_ad1, tdt_b1),
              (tdt_W2, tdt_as2, tdt_ad2, tdt_b2)))):
        srcA = jnp.concatenate([ei[0], loop, pad])
        dstA = jnp.concatenate([ei[1], loop, pad])
        pkA = srcA | (dstA << 14)
        act = hT
        for (W, a_s, a_d, b) in params:
            act = _gat_layer(act, W, a_s, a_d, b, pkA, p2)
        acts[len(acts)] = act

    semp = jnp.pad(sem[None, :], ((0, 0), (0, 126)))
    b1p = jnp.pad(cls_b1[None, :], ((0, 0), (0, 64)))
    W2t8 = jnp.pad(cls_W2.T, ((0, 6), (0, 64)))
    b2p = jnp.pad(cls_b2[None, :], ((0, 0), (0, 126)))
    lg8 = _cls(acts[0], acts[1], semp, cls_W1.T, b1p, W2t8, b2p)
    return lg8[:2, :n].T


# speedup vs baseline: 1.0021x; 1.0021x over previous
import functools

import jax
import jax.numpy as jnp
from jax import lax
from jax.experimental import pallas as pl
from jax.experimental.pallas import tpu as pltpu
from jax.experimental.pallas import tpu_sc as plsc

N = 10000
HID = 128
H = 8
C = 16
NP = 10240
NBLK = NP // 256
NW = 32
K1 = 1296
K2 = 2592

@functools.lru_cache(maxsize=1)
def _mesh():
    return plsc.VectorSubcoreMesh(
        core_axis_name="c", subcore_axis_name="s",
        num_cores=2, num_subcores=16)


def _f32(shape):
    return jax.ShapeDtypeStruct(shape, jnp.float32)


def _proj_body(xt, wt, pb, out):
    out[...] = jnp.dot(wt[...], xt[...], preferred_element_type=jnp.float32) \
        + pb[...].T


def _proj(xT, Wt, pb2d):
    return pl.pallas_call(
        _proj_body,
        grid=(NBLK,),
        in_specs=[
            pl.BlockSpec((HID, 256), lambda i: (0, i)),
            pl.BlockSpec((HID, HID), lambda i: (0, 0)),
            pl.BlockSpec((1, HID), lambda i: (0, 0)),
        ],
        out_specs=pl.BlockSpec((HID, 256), lambda i: (0, i)),
        out_shape=_f32((HID, NP)),
    )(xT, Wt, pb2d)


def _tc1_body(act, wt, asb, adb, xwt, asrc, adst, ms, md):
    i = pl.program_id(0)
    a = act[...]
    xw = jnp.dot(wt[...], a, preferred_element_type=jnp.float32)
    xwt[...] = xw
    s = jnp.dot(asb[...], xw, preferred_element_type=jnp.float32)
    d = jnp.dot(adb[...], xw, preferred_element_type=jnp.float32)
    asrc[...] = s
    adst[...] = d
    neg = jnp.full((H, 128), -jnp.inf, jnp.float32)
    pms = jnp.where(i == 0, neg, ms[...])
    pmd = jnp.where(i == 0, neg, md[...])
    ms[...] = jnp.maximum(pms, jnp.max(s, axis=1)[:, None])
    md[...] = jnp.maximum(pmd, jnp.max(d, axis=1)[:, None])


def _tc1(actT, Wt, As, Ad):
    return pl.pallas_call(
        _tc1_body,
        grid=(NBLK,),
        in_specs=[
            pl.BlockSpec((HID, 256), lambda i: (0, i)),
            pl.BlockSpec((HID, HID), lambda i: (0, 0)),
            pl.BlockSpec((H, HID), lambda i: (0, 0)),
            pl.BlockSpec((H, HID), lambda i: (0, 0)),
        ],
        out_specs=[
            pl.BlockSpec((HID, 256), lambda i: (0, i)),
            pl.BlockSpec((H, 256), lambda i: (0, i)),
            pl.BlockSpec((H, 256), lambda i: (0, i)),
            pl.BlockSpec((H, 128), lambda i: (0, 0)),
            pl.BlockSpec((H, 128), lambda i: (0, 0)),
        ],
        out_shape=[_f32((HID, NP)), _f32((H, NP)), _f32((H, NP)),
                   _f32((H, 128)), _f32((H, 128))],
    )(actT, Wt, As, Ad)


def _cls_body(tut, tdt, semp, w1t, b1p, w2t, b2p, out):
    s0 = semp[0, 0]
    s1 = semp[0, 1]
    m = jnp.maximum(s0, s1)
    e0 = jnp.exp(s0 - m)
    e1 = jnp.exp(s1 - m)
    w0 = e0 / (e0 + e1)
    w1 = e1 / (e0 + e1)
    z = w0 * tut[...] + w1 * tdt[...]
    hh = jnp.dot(w1t[...], z, preferred_element_type=jnp.float32) \
        + b1p[...][:, :64].T
    hh = jnp.maximum(hh, 0.0)
    lg = jnp.dot(w2t[...][:, :64], hh, preferred_element_type=jnp.float32) \
        + b2p[...][:, :8].T
    out[...] = lg


def _cls(tutT, tdtT, semp, W1t, b1p, W2t8, b2p):
    return pl.pallas_call(
        _cls_body,
        grid=(NBLK,),
        in_specs=[
            pl.BlockSpec((HID, 256), lambda i: (0, i)),
            pl.BlockSpec((HID, 256), lambda i: (0, i)),
            pl.BlockSpec((1, 128), lambda i: (0, 0)),
            pl.BlockSpec((64, 128), lambda i: (0, 0)),
            pl.BlockSpec((1, 128), lambda i: (0, 0)),
            pl.BlockSpec((8, 128), lambda i: (0, 0)),
            pl.BlockSpec((1, 128), lambda i: (0, 0)),
        ],
        out_specs=pl.BlockSpec((8, 256), lambda i: (0, i)),
        out_shape=_f32((8, NP)),
    )(tutT, tdtT, semp, W1t, b1p, W2t8, b2p)


def _splat_i32(v):
    return jnp.zeros((16,), jnp.int32) + v


def _p2_builder(Ep):
    nch = Ep // K2

    def body(pkH, asrcF, adstF, g16H, xwFH, biasH, outFH,
             xw4, out4, asv, adv, denv, bv, pkb, g16v, sem0, sem1):
        cid = lax.axis_index("c")
        sid = lax.axis_index("s")
        wid = sid * 2 + cid
        hd = wid // 4
        pltpu.sync_copy(xwFH.at[pl.ds(4 * wid * NP, 4 * NP)], xw4)
        pltpu.sync_copy(asrcF.at[pl.ds(hd * NP, NP)], asv)
        pltpu.sync_copy(adstF.at[pl.ds(hd * NP, NP)], adv)
        pltpu.sync_copy(biasH, bv)
        pltpu.sync_copy(g16H, g16v)
        iota = lax.iota(jnp.int32, 16)
        gh = plsc.load_gather(g16v, [_splat_i32(0) + hd])

        @plsc.parallel_loop(0, 4 * NP // 16, 1, unroll=8)
        def _(t):
            idx = _splat_i32(t * 16) + iota
            plsc.store_scatter(out4, [idx], jnp.zeros((16,), jnp.float32))

        @plsc.parallel_loop(0, NP // 16, 1, unroll=8)
        def _(t):
            idx = _splat_i32(t * 16) + iota
            plsc.store_scatter(denv, [idx], jnp.zeros((16,), jnp.float32))

        sems = (sem0, sem1)

        def issue(ci, b):
            pltpu.async_copy(pkH.at[pl.ds(ci * K2, K2)], pkb.at[b], sems[b])

        issue(0, 0)
        issue(1, 1)

        def obody(i2, _):
            for b in (0, 1):
                ci = i2 * 2 + b
                pltpu.make_async_copy(
                    pkH.at[pl.ds(ci * K2, K2)], pkb.at[b], sems[b]).wait()
                bvec = _splat_i32(b)

                @plsc.parallel_loop(0, K2 // 16, 1, unroll=2)
                def _(g):
                    lidx = _splat_i32(g * 16) + iota
                    pk = plsc.load_gather(pkb, [bvec, lidx])
                    sv = pk & 16383
                    dv = pk >> 14
                    al = plsc.load_gather(asv, [sv]) \
                        + plsc.load_gather(adv, [dv])
                    al = jnp.maximum(al, 0.2 * al) - gh
                    ex = jnp.exp(al)
                    plsc.addupdate_scatter(denv, [dv], ex)
                    for c in range(4):
                        cN = _splat_i32(c * NP)
                        xg = plsc.load_gather(xw4, [cN + sv])
                        plsc.addupdate_scatter(out4, [cN + dv], xg * ex)

                @pl.when(ci + 2 < nch)
                def _():
                    issue(ci + 2, b)
            return 0
        lax.fori_loop(0, nch // 2, obody, 0)

        for c in range(4):
            bcv = plsc.load_gather(bv, [_splat_i32(0) + 4 * wid + c])

            @plsc.parallel_loop(0, NP // 16, 1, unroll=8)
            def _(t):
                tidx = _splat_i32(t * 16) + iota
                idx = _splat_i32(c * NP) + tidx
                den = plsc.load_gather(denv, [tidx]) + 1e-16
                v = plsc.load_gather(out4, [idx]) / den + bcv
                act = jnp.where(v > 0, v, jnp.exp(v) - 1.0)
                plsc.store_scatter(out4, [idx], act)
        pltpu.sync_copy(out4, outFH.at[pl.ds(4 * wid * NP, 4 * NP)])

    return pl.kernel(
        body,
        out_type=_f32((HID * NP,)),
        mesh=_mesh(),
        compiler_params=pltpu.CompilerParams(
            use_tc_tiling_on_sc=False, needs_layout_passes=False),
        scratch_types=[
            pltpu.VMEM((4 * NP,), jnp.float32),
            pltpu.VMEM((4 * NP,), jnp.float32),
            pltpu.VMEM((NP,), jnp.float32),
            pltpu.VMEM((NP,), jnp.float32),
            pltpu.VMEM((NP,), jnp.float32),
            pltpu.VMEM((HID,), jnp.float32),
            pltpu.VMEM((2, K2), jnp.int32),
            pltpu.VMEM((16,), jnp.float32),
            pltpu.SemaphoreType.DMA,
            pltpu.SemaphoreType.DMA,
        ],
    )


def _blockdiag(att):
    return (att[:, None, :] * jnp.eye(H, dtype=att.dtype)[:, :, None]
            ).reshape(H, H * C)


def _gat_layer(actT, W, att_s, att_d, bias, pkA, p2):
    xwT, asrcT, adstT, mS, mD = _tc1(
        actT, W.T, _blockdiag(att_s), _blockdiag(att_d))
    sm = mS[:, 0] + mD[:, 0]
    g16 = jnp.concatenate([jnp.maximum(sm, 0.2 * sm)] * 2)
    outF = p2(pkA, asrcT.reshape(-1), adstT.reshape(-1), g16,
              xwT.reshape(-1), bias)
    return outF.reshape(HID, NP)


def kernel(x, proj_W, proj_b,
           tut_W1, tut_as1, tut_ad1, tut_b1, tut_W2, tut_as2, tut_ad2, tut_b2,
           tdt_W1, tdt_as1, tdt_ad1, tdt_b1, tdt_W2, tdt_as2, tdt_ad2, tdt_b2,
           sem, cls_W1, cls_b1, cls_W2, cls_b2,
           tut_edge_index, tdt_edge_index):
    n = x.shape[0]
    e = tut_edge_index.shape[1]
    etot = e + n
    npt = -(-etot // (NW * K1)) * K1
    Ep = NW * npt

    xT = jnp.pad(x.T, ((0, 0), (0, NP - n)))
    hT = _proj(xT, proj_W.T, proj_b[None, :])

    p2 = _p2_builder(Ep)

    loop = jnp.arange(n, dtype=jnp.int32)
    pad = jnp.full((Ep - etot,), n, jnp.int32)

    acts = {}
    for ei, params in (
            (tut_edge_index,
             ((tut_W1, tut_as1, tut_ad1, tut_b1),
              (tut_W2, tut_as2, tut_ad2, tut_b2))),
            (tdt_edge_index,
             ((tdt_W1, tdt_as1, tdt_ad1, tdt_b1),
              (tdt_W2, tdt_as2, tdt_ad2, tdt_b2)))):
        srcA = jnp.concatenate([ei[0], loop, pad])
        dstA = jnp.concatenate([ei[1], loop, pad])
        pkA = srcA | (dstA << 14)
        act = hT
        for (W, a_s, a_d, b) in params:
            act = _gat_layer(act, W, a_s, a_d, b, pkA, p2)
        acts[len(acts)] = act

    semp = jnp.pad(sem[None, :], ((0, 0), (0, 126)))
    b1p = jnp.pad(cls_b1[None, :], ((0, 0), (0, 64)))
    W2t8 = jnp.pad(cls_W2.T, ((0, 6), (0, 64)))
    b2p = jnp.pad(cls_b2[None, :], ((0, 0), (0, 126)))
    lg8 = _cls(acts[0], acts[1], semp, cls_W1.T, b1p, W2t8, b2p)
    return lg8[:2, :n].T
